# Initial kernel scaffold; baseline (speedup 1.0000x reference)
import jax
import jax.numpy as jnp
from jax.experimental import pallas as pl


def kernel(clf_proba, reg_preds_all, all_proposal_boxes, device):
    raise NotImplementedError("write your pallas kernel here")


# speedup vs baseline: 22.1789x; 22.1789x over previous
import functools

import jax
import jax.numpy as jnp
from jax import lax
from jax.experimental import pallas as pl
from jax.experimental.pallas import tpu as pltpu
from jax.experimental.pallas import tpu_sc as plsc

_CONF = 0.01
_NMS_THR = 0.5
_TOP_K = 10
_LEN_THR = 3.0
_LO, _HI = 0.0, 416.0

_B = 16
_N = 20000
_CHUNK = 10000
_CAP = _N + 32


def _vf(x):
    return jnp.full((16,), x, jnp.float32)


def _vi(x):
    return jnp.full((16,), x, jnp.int32)


def _nms_body(s_hbm, dx_hbm, dw_hbm, p1_hbm, p2_hbm, out_hbm,
              bs, bdx, bdw, b1, b2, cs, c1, c2, det):
    cid = lax.axis_index("c")
    sid = lax.axis_index("s")
    wid = sid * 2 + cid

    @pl.when(wid < _B)
    def _work():
        b = wid
        lanes = lax.iota(jnp.int32, 16)
        neg1 = _vf(-1.0)
        det[pl.ds(0, 16)] = neg1
        det[pl.ds(16, 16)] = neg1

        def stage_and_compact(ch, cnt):
            off = b * _N + ch * _CHUNK
            pltpu.sync_copy(s_hbm.at[pl.ds(off, _CHUNK)], bs)
            pltpu.sync_copy(dx_hbm.at[pl.ds(off, _CHUNK)], bdx)
            pltpu.sync_copy(dw_hbm.at[pl.ds(off, _CHUNK)], bdw)
            pltpu.sync_copy(p1_hbm.at[pl.ds(off, _CHUNK)], b1)
            pltpu.sync_copy(p2_hbm.at[pl.ds(off, _CHUNK)], b2)

            def body(j, cnt):
                sl = pl.ds(j * 16, 16)
                sv = bs[sl]
                w = b2[sl] - b1[sl]
                ctr = b1[sl] + _vf(0.5) * w
                pc = ctr + bdx[sl] * w
                pw = jnp.exp(bdw[sl]) * w
                hw = _vf(0.5) * pw
                x1 = jnp.minimum(jnp.maximum(pc - hw, _vf(_LO)), _vf(_HI))
                x2 = jnp.minimum(jnp.maximum(pc + hw, _vf(_LO)), _vf(_HI))
                m = (sv > _vf(_CONF)) & ((x2 - x1) > _vf(_LEN_THR))
                csum = plsc.cumsum(m.astype(jnp.int32))
                idx = _vi(cnt) + csum - _vi(1)
                plsc.store_scatter(cs, [idx], sv, mask=m)
                plsc.store_scatter(c1, [idx], x1, mask=m)
                plsc.store_scatter(c2, [idx], x2, mask=m)
                return cnt + jnp.max(csum)

            return lax.fori_loop(0, _CHUNK // 16, body, cnt)

        cnt = jnp.int32(0)
        for ch in range(_N // _CHUNK):
            cnt = stage_and_compact(ch, cnt)

        plsc.store_scatter(cs, [_vi(cnt) + lanes], neg1)
        nsteps = (cnt + 15) // 16

        def pick(t, _):
            def amax(j, st):
                bv, bi = st
                sv = cs[pl.ds(j * 16, 16)]
                iv = _vi(j * 16) + lanes
                upd = sv > bv
                return jnp.where(upd, sv, bv), jnp.where(upd, iv, bi)

            bv, bi = lax.fori_loop(0, nsteps, amax, (neg1, _vi(0)))
            mx = jnp.max(bv)
            cand = jnp.where(bv == _vf(mx), bi, _vi(2 ** 30))
            k = jnp.min(cand)

            @pl.when(mx > 0.0)
            def _emit():
                kv = _vi(k)
                x1kv = plsc.load_gather(c1, [kv])
                x2kv = plsc.load_gather(c2, [kv])
                skv = plsc.load_gather(cs, [kv])
                val = jnp.where(lanes == _vi(0), x1kv,
                                jnp.where(lanes == _vi(1), x2kv, skv))
                plsc.store_scatter(det, [_vi(3 * t) + lanes], val,
                                   mask=lanes < _vi(3))
                lkv = x2kv - x1kv

                def suppress(j, _):
                    sl = pl.ds(j * 16, 16)
                    a1 = c1[sl]
                    a2 = c2[sl]
                    inter = jnp.maximum(
                        jnp.minimum(x2kv, a2) - jnp.maximum(x1kv, a1),
                        _vf(0.0))
                    union = lkv + (a2 - a1) - inter
                    iou = inter / jnp.maximum(union, _vf(1e-12))
                    iv = _vi(j * 16) + lanes
                    kill = (iou > _vf(_NMS_THR)) | (iv == kv)
                    cs[sl] = jnp.where(kill, neg1, cs[sl])
                    return 0

                lax.fori_loop(0, nsteps, suppress, 0)

            return 0

        lax.fori_loop(0, _TOP_K, pick, 0)
        pltpu.sync_copy(det, out_hbm.at[pl.ds(b * 32, 32)])


_sc_nms = functools.partial(
    pl.kernel,
    out_type=jax.ShapeDtypeStruct((_B * 32,), jnp.float32),
    mesh=plsc.VectorSubcoreMesh(core_axis_name="c", subcore_axis_name="s",
                                num_cores=2, num_subcores=16),
    scratch_types=[
        pltpu.VMEM((_CHUNK,), jnp.float32),
        pltpu.VMEM((_CHUNK,), jnp.float32),
        pltpu.VMEM((_CHUNK,), jnp.float32),
        pltpu.VMEM((_CHUNK,), jnp.float32),
        pltpu.VMEM((_CHUNK,), jnp.float32),
        pltpu.VMEM((_CAP,), jnp.float32),
        pltpu.VMEM((_CAP,), jnp.float32),
        pltpu.VMEM((_CAP,), jnp.float32),
        pltpu.VMEM((32,), jnp.float32),
    ],
    compiler_params=pltpu.CompilerParams(needs_layout_passes=False),
)(_nms_body)


@jax.jit
def kernel(clf_proba, reg_preds_all, all_proposal_boxes, device):
    del device
    scores = clf_proba[:, :, 0].reshape(-1)
    dx = reg_preds_all[:, :, 0].reshape(-1)
    dw = reg_preds_all[:, :, 1].reshape(-1)
    p1 = all_proposal_boxes[:, :, 0].reshape(-1)
    p2 = all_proposal_boxes[:, :, 1].reshape(-1)
    out = _sc_nms(scores, dx, dw, p1, p2)
    return out.reshape(_B, 32)[:, :_TOP_K * 3].reshape(_B, _TOP_K, 3)


# speedup vs baseline: 31.9960x; 1.4426x over previous
import functools

import jax
import jax.numpy as jnp
from jax import lax
from jax.experimental import pallas as pl
from jax.experimental.pallas import tpu as pltpu
from jax.experimental.pallas import tpu_sc as plsc

_CONF = 0.01
_NMS_THR = 0.5
_TOP_K = 10
_LEN_THR = 3.0
_LO, _HI = 0.0, 416.0

_B = 16
_N = 20000
_CH = 4000
_NCH = _N // _CH
_CAP = _N + 32
_QUADS = _CH // 64
_REM_GROUPS = (_CH - _QUADS * 64) // 16


def _vf(x):
    return jnp.full((16,), x, jnp.float32)


def _vi(x):
    return jnp.full((16,), x, jnp.int32)


def _nms_body(s_hbm, dx_hbm, dw_hbm, p1_hbm, p2_hbm, out_hbm,
              bs0, bdx0, bdw0, b10, b20,
              bs1, bdx1, bdw1, b11, b21,
              cs, c1, c2, det, sem0, sem1):
    cid = lax.axis_index("c")
    sid = lax.axis_index("s")
    wid = sid * 2 + cid

    bufs = ((bs0, bdx0, bdw0, b10, b20), (bs1, bdx1, bdw1, b11, b21))
    sems = (sem0, sem1)

    @pl.when(wid < _B)
    def _work():
        b = wid
        lanes = lax.iota(jnp.int32, 16)
        neg1 = _vf(-1.0)
        det[pl.ds(0, 16)] = neg1
        det[pl.ds(16, 16)] = neg1

        def issue(ch, slot):
            off = b * _N + ch * _CH
            srcs = (s_hbm, dx_hbm, dw_hbm, p1_hbm, p2_hbm)
            return [
                pltpu.async_copy(src.at[pl.ds(off, _CH)], dst, sems[slot])
                for src, dst in zip(srcs, bufs[slot])
            ]

        def group(bufset, goff, cntv):
            bsb, bdxb, bdwb, b1b, b2b = bufset
            sl = pl.ds(goff, 16)
            sv = bsb[sl]
            w = b2b[sl] - b1b[sl]
            ctr = b1b[sl] + _vf(0.5) * w
            pc = ctr + bdxb[sl] * w
            hw = _vf(0.5) * (jnp.exp(bdwb[sl]) * w)
            x1 = jnp.minimum(jnp.maximum(pc - hw, _vf(_LO)), _vf(_HI))
            x2 = jnp.minimum(jnp.maximum(pc + hw, _vf(_LO)), _vf(_HI))
            m = (sv > _vf(_CONF)) & ((x2 - x1) > _vf(_LEN_THR))
            csum = plsc.cumsum(m.astype(jnp.int32))
            idx = cntv + csum - _vi(1)
            plsc.store_scatter(cs, [idx], sv, mask=m)
            plsc.store_scatter(c1, [idx], x1, mask=m)
            plsc.store_scatter(c2, [idx], x2, mask=m)
            return cntv + plsc.all_reduce_population_count(m)

        def quad_body(bufset, base, cntv):
            bsb, bdxb, bdwb, b1b, b2b = bufset
            sls = [pl.ds(base + u * 16, 16) for u in range(4)]
            svs = [bsb[sl] for sl in sls]
            dxs = [bdxb[sl] for sl in sls]
            dws = [bdwb[sl] for sl in sls]
            p1s = [b1b[sl] for sl in sls]
            p2s = [b2b[sl] for sl in sls]
            es = [jnp.exp(dw) for dw in dws]
            ws = [p2 - p1 for p1, p2 in zip(p1s, p2s)]
            ctrs = [p1 + _vf(0.5) * w for p1, w in zip(p1s, ws)]
            pcs = [ctr + dx * w for ctr, dx, w in zip(ctrs, dxs, ws)]
            hws = [_vf(0.5) * (e * w) for e, w in zip(es, ws)]
            x1s = [jnp.minimum(jnp.maximum(pc - hw, _vf(_LO)), _vf(_HI))
                   for pc, hw in zip(pcs, hws)]
            x2s = [jnp.minimum(jnp.maximum(pc + hw, _vf(_LO)), _vf(_HI))
                   for pc, hw in zip(pcs, hws)]
            ms = [(sv > _vf(_CONF)) & ((x2 - x1) > _vf(_LEN_THR))
                  for sv, x1, x2 in zip(svs, x1s, x2s)]
            csums = [plsc.cumsum(m.astype(jnp.int32)) for m in ms]
            pops = [plsc.all_reduce_population_count(m) for m in ms]
            cnts = [cntv]
            for u in range(3):
                cnts.append(cnts[u] + pops[u])
            idxs = [cnts[u] + csums[u] - _vi(1) for u in range(4)]
            for u in range(4):
                plsc.store_scatter(cs, [idxs[u]], svs[u], mask=ms[u])
                plsc.store_scatter(c1, [idxs[u]], x1s[u], mask=ms[u])
                plsc.store_scatter(c2, [idxs[u]], x2s[u], mask=ms[u])
            return cnts[3] + pops[3]

        cntv = _vi(0)
        descs = issue(0, 0)
        for ch in range(_NCH):
            slot = ch % 2
            nxt = issue(ch + 1, 1 - slot) if ch + 1 < _NCH else None
            for d in descs:
                d.wait()
            bufset = bufs[slot]

            def quad(q, cntv, bufset=bufset):
                return quad_body(bufset, q * 64, cntv)

            cntv = lax.fori_loop(0, _QUADS, quad, cntv)
            for u in range(_REM_GROUPS):
                cntv = group(bufset, _QUADS * 64 + u * 16, cntv)
            descs = nxt

        cnt = jnp.max(cntv)
        plsc.store_scatter(cs, [_vi(cnt) + lanes], neg1)
        nsteps = (cnt + 15) // 16

        def pick(t, carry):
            kprev, x1p, x2p = carry
            kpv = _vi(kprev)
            x1pv = _vf(x1p)
            x2pv = _vf(x2p)
            lpv = x2pv - x1pv

            def fused(j, st):
                bv, bi = st
                sl = pl.ds(j * 16, 16)
                sv = cs[sl]
                a1 = c1[sl]
                a2 = c2[sl]
                inter = jnp.maximum(
                    jnp.minimum(x2pv, a2) - jnp.maximum(x1pv, a1), _vf(0.0))
                union = lpv + (a2 - a1) - inter
                iou = inter / jnp.maximum(union, _vf(1e-12))
                iv = _vi(j * 16) + lanes
                kill = (iou > _vf(_NMS_THR)) | (iv == kpv)
                sv = jnp.where(kill, neg1, sv)
                cs[sl] = sv
                upd = sv > bv
                return jnp.where(upd, sv, bv), jnp.where(upd, iv, bi)

            bv, bi = lax.fori_loop(0, nsteps, fused, (neg1, _vi(0)))
            mx = jnp.max(bv)
            cand = jnp.where(bv == _vf(mx), bi, _vi(2 ** 30))
            k = jnp.min(cand)

            def emit():
                kv = _vi(k)
                x1kv = plsc.load_gather(c1, [kv])
                x2kv = plsc.load_gather(c2, [kv])
                skv = plsc.load_gather(cs, [kv])
                val = jnp.where(lanes == _vi(0), x1kv,
                                jnp.where(lanes == _vi(1), x2kv, skv))
                plsc.store_scatter(det, [_vi(3 * t) + lanes], val,
                                   mask=lanes < _vi(3))
                return k, jnp.max(x1kv), jnp.max(x2kv)

            def skip():
                return jnp.int32(-1), jnp.float32(0.0), jnp.float32(0.0)

            return lax.cond(mx > 0.0, emit, skip)

        lax.fori_loop(0, _TOP_K, pick,
                      (jnp.int32(-1), jnp.float32(0.0), jnp.float32(0.0)))
        pltpu.sync_copy(det, out_hbm.at[pl.ds(b * 32, 32)])


_sc_nms = functools.partial(
    pl.kernel,
    out_type=jax.ShapeDtypeStruct((_B * 32,), jnp.float32),
    mesh=plsc.VectorSubcoreMesh(core_axis_name="c", subcore_axis_name="s",
                                num_cores=2, num_subcores=16),
    scratch_types=[
        pltpu.VMEM((_CH,), jnp.float32),
        pltpu.VMEM((_CH,), jnp.float32),
        pltpu.VMEM((_CH,), jnp.float32),
        pltpu.VMEM((_CH,), jnp.float32),
        pltpu.VMEM((_CH,), jnp.float32),
        pltpu.VMEM((_CH,), jnp.float32),
        pltpu.VMEM((_CH,), jnp.float32),
        pltpu.VMEM((_CH,), jnp.float32),
        pltpu.VMEM((_CH,), jnp.float32),
        pltpu.VMEM((_CH,), jnp.float32),
        pltpu.VMEM((_CAP,), jnp.float32),
        pltpu.VMEM((_CAP,), jnp.float32),
        pltpu.VMEM((_CAP,), jnp.float32),
        pltpu.VMEM((32,), jnp.float32),
        pltpu.SemaphoreType.DMA,
        pltpu.SemaphoreType.DMA,
    ],
    compiler_params=pltpu.CompilerParams(needs_layout_passes=False),
)(_nms_body)


@jax.jit
def kernel(clf_proba, reg_preds_all, all_proposal_boxes, device):
    del device
    scores = clf_proba[:, :, 0].reshape(-1)
    dx = reg_preds_all[:, :, 0].reshape(-1)
    dw = reg_preds_all[:, :, 1].reshape(-1)
    p1 = all_proposal_boxes[:, :, 0].reshape(-1)
    p2 = all_proposal_boxes[:, :, 1].reshape(-1)
    out = _sc_nms(scores, dx, dw, p1, p2)
    return out.reshape(_B, 32)[:, :_TOP_K * 3].reshape(_B, _TOP_K, 3)


# speedup vs baseline: 32.8365x; 1.0263x over previous
import functools

import jax
import jax.numpy as jnp
from jax import lax
from jax.experimental import pallas as pl
from jax.experimental.pallas import tpu as pltpu
from jax.experimental.pallas import tpu_sc as plsc

_CONF = 0.01
_NMS_THR = 0.5
_TOP_K = 10
_LEN_THR = 3.0
_LO, _HI = 0.0, 416.0

_B = 16
_N = 20000
_CH = 4000
_NCH = _N // _CH
_CAP = _N + 32
_QUADS = _CH // 64
_REM_GROUPS = (_CH - _QUADS * 64) // 16


def _vf(x):
    return jnp.full((16,), x, jnp.float32)


def _vi(x):
    return jnp.full((16,), x, jnp.int32)


def _nms_body(s_hbm, dx_hbm, dw_hbm, p1_hbm, p2_hbm, out_hbm,
              bs0, bdx0, bdw0, b10, b20,
              bs1, bdx1, bdw1, b11, b21,
              cs, c1, c2, det, sem0, sem1):
    sid = lax.axis_index("s")
    wid = sid

    bufs = ((bs0, bdx0, bdw0, b10, b20), (bs1, bdx1, bdw1, b11, b21))
    sems = (sem0, sem1)

    @pl.when(wid < _B)
    def _work():
        b = wid
        lanes = lax.iota(jnp.int32, 16)
        neg1 = _vf(-1.0)
        det[pl.ds(0, 16)] = neg1
        det[pl.ds(16, 16)] = neg1

        def issue(ch, slot):
            off = b * _N + ch * _CH
            srcs = (s_hbm, dx_hbm, dw_hbm, p1_hbm, p2_hbm)
            return [
                pltpu.async_copy(src.at[pl.ds(off, _CH)], dst, sems[slot])
                for src, dst in zip(srcs, bufs[slot])
            ]

        def group(bufset, goff, cntv):
            bsb, bdxb, bdwb, b1b, b2b = bufset
            sl = pl.ds(goff, 16)
            sv = bsb[sl]
            w = b2b[sl] - b1b[sl]
            ctr = b1b[sl] + _vf(0.5) * w
            pc = ctr + bdxb[sl] * w
            hw = _vf(0.5) * (jnp.exp(bdwb[sl]) * w)
            x1 = jnp.minimum(jnp.maximum(pc - hw, _vf(_LO)), _vf(_HI))
            x2 = jnp.minimum(jnp.maximum(pc + hw, _vf(_LO)), _vf(_HI))
            m = (sv > _vf(_CONF)) & ((x2 - x1) > _vf(_LEN_THR))
            csum = plsc.cumsum(m.astype(jnp.int32))
            idx = cntv + csum - _vi(1)
            plsc.store_scatter(cs, [idx], sv, mask=m)
            plsc.store_scatter(c1, [idx], x1, mask=m)
            plsc.store_scatter(c2, [idx], x2, mask=m)
            return cntv + plsc.all_reduce_population_count(m)

        def quad_body(bufset, base, cntv):
            bsb, bdxb, bdwb, b1b, b2b = bufset
            sls = [pl.ds(base + u * 16, 16) for u in range(4)]
            svs = [bsb[sl] for sl in sls]
            dxs = [bdxb[sl] for sl in sls]
            dws = [bdwb[sl] for sl in sls]
            p1s = [b1b[sl] for sl in sls]
            p2s = [b2b[sl] for sl in sls]
            es = [jnp.exp(dw) for dw in dws]
            ws = [p2 - p1 for p1, p2 in zip(p1s, p2s)]
            ctrs = [p1 + _vf(0.5) * w for p1, w in zip(p1s, ws)]
            pcs = [ctr + dx * w for ctr, dx, w in zip(ctrs, dxs, ws)]
            hws = [_vf(0.5) * (e * w) for e, w in zip(es, ws)]
            x1s = [jnp.minimum(jnp.maximum(pc - hw, _vf(_LO)), _vf(_HI))
                   for pc, hw in zip(pcs, hws)]
            x2s = [jnp.minimum(jnp.maximum(pc + hw, _vf(_LO)), _vf(_HI))
                   for pc, hw in zip(pcs, hws)]
            ms = [(sv > _vf(_CONF)) & ((x2 - x1) > _vf(_LEN_THR))
                  for sv, x1, x2 in zip(svs, x1s, x2s)]
            csums = [plsc.cumsum(m.astype(jnp.int32)) for m in ms]
            pops = [plsc.all_reduce_population_count(m) for m in ms]
            cnts = [cntv]
            for u in range(3):
                cnts.append(cnts[u] + pops[u])
            idxs = [cnts[u] + csums[u] - _vi(1) for u in range(4)]
            for u in range(4):
                plsc.store_scatter(cs, [idxs[u]], svs[u], mask=ms[u])
                plsc.store_scatter(c1, [idxs[u]], x1s[u], mask=ms[u])
                plsc.store_scatter(c2, [idxs[u]], x2s[u], mask=ms[u])
            return cnts[3] + pops[3]

        cntv = _vi(0)
        descs = issue(0, 0)
        for ch in range(_NCH):
            slot = ch % 2
            nxt = issue(ch + 1, 1 - slot) if ch + 1 < _NCH else None
            for d in descs:
                d.wait()
            bufset = bufs[slot]

            def quad(q, cntv, bufset=bufset):
                return quad_body(bufset, q * 64, cntv)

            cntv = lax.fori_loop(0, _QUADS, quad, cntv)
            for u in range(_REM_GROUPS):
                cntv = group(bufset, _QUADS * 64 + u * 16, cntv)
            descs = nxt

        cnt = jnp.max(cntv)
        plsc.store_scatter(cs, [_vi(cnt) + lanes], neg1)
        nsteps = (cnt + 15) // 16

        def pick(t, carry):
            kprev, x1p, x2p = carry
            kpv = _vi(kprev)
            x1pv = _vf(x1p)
            x2pv = _vf(x2p)
            lpv = x2pv - x1pv

            def fused(j, st):
                bv, bi = st
                sl = pl.ds(j * 16, 16)
                sv = cs[sl]
                a1 = c1[sl]
                a2 = c2[sl]
                inter = jnp.maximum(
                    jnp.minimum(x2pv, a2) - jnp.maximum(x1pv, a1), _vf(0.0))
                union = lpv + (a2 - a1) - inter
                iou = inter / jnp.maximum(union, _vf(1e-12))
                iv = _vi(j * 16) + lanes
                kill = (iou > _vf(_NMS_THR)) | (iv == kpv)
                sv = jnp.where(kill, neg1, sv)
                cs[sl] = sv
                upd = sv > bv
                return jnp.where(upd, sv, bv), jnp.where(upd, iv, bi)

            bv, bi = lax.fori_loop(0, nsteps, fused, (neg1, _vi(0)))
            mx = jnp.max(bv)
            cand = jnp.where(bv == _vf(mx), bi, _vi(2 ** 30))
            k = jnp.min(cand)

            def emit():
                kv = _vi(k)
                x1kv = plsc.load_gather(c1, [kv])
                x2kv = plsc.load_gather(c2, [kv])
                skv = plsc.load_gather(cs, [kv])
                val = jnp.where(lanes == _vi(0), x1kv,
                                jnp.where(lanes == _vi(1), x2kv, skv))
                plsc.store_scatter(det, [_vi(3 * t) + lanes], val,
                                   mask=lanes < _vi(3))
                return k, jnp.max(x1kv), jnp.max(x2kv)

            def skip():
                return jnp.int32(-1), jnp.float32(0.0), jnp.float32(0.0)

            return lax.cond(mx > 0.0, emit, skip)

        lax.fori_loop(0, _TOP_K, pick,
                      (jnp.int32(-1), jnp.float32(0.0), jnp.float32(0.0)))
        pltpu.sync_copy(det, out_hbm.at[pl.ds(b * 32, 32)])


_sc_nms = functools.partial(
    pl.kernel,
    out_type=jax.ShapeDtypeStruct((_B * 32,), jnp.float32),
    mesh=plsc.VectorSubcoreMesh(core_axis_name="c", subcore_axis_name="s",
                                num_cores=1, num_subcores=16),
    scratch_types=[
        pltpu.VMEM((_CH,), jnp.float32),
        pltpu.VMEM((_CH,), jnp.float32),
        pltpu.VMEM((_CH,), jnp.float32),
        pltpu.VMEM((_CH,), jnp.float32),
        pltpu.VMEM((_CH,), jnp.float32),
        pltpu.VMEM((_CH,), jnp.float32),
        pltpu.VMEM((_CH,), jnp.float32),
        pltpu.VMEM((_CH,), jnp.float32),
        pltpu.VMEM((_CH,), jnp.float32),
        pltpu.VMEM((_CH,), jnp.float32),
        pltpu.VMEM((_CAP,), jnp.float32),
        pltpu.VMEM((_CAP,), jnp.float32),
        pltpu.VMEM((_CAP,), jnp.float32),
        pltpu.VMEM((32,), jnp.float32),
        pltpu.SemaphoreType.DMA,
        pltpu.SemaphoreType.DMA,
    ],
    compiler_params=pltpu.CompilerParams(needs_layout_passes=False),
)(_nms_body)


@jax.jit
def kernel(clf_proba, reg_preds_all, all_proposal_boxes, device):
    del device
    scores = clf_proba[:, :, 0].reshape(-1)
    dx = reg_preds_all[:, :, 0].reshape(-1)
    dw = reg_preds_all[:, :, 1].reshape(-1)
    p1 = all_proposal_boxes[:, :, 0].reshape(-1)
    p2 = all_proposal_boxes[:, :, 1].reshape(-1)
    out = _sc_nms(scores, dx, dw, p1, p2)
    return out.reshape(_B, 32)[:, :_TOP_K * 3].reshape(_B, _TOP_K, 3)
